# baseline (device time: 18467 ns/iter reference)
import os

import jax
import jax.numpy as jnp
from jax import lax
from jax.experimental import pallas as pl
from jax.experimental.pallas import tpu as pltpu

_PROBE = os.environ.get("KERNEL_PROBE", "")


def kernel(Q, K, V):
    b, s, h, d = Q.shape
    hd = h * d
    da = 2 * d
    scale = d ** -0.5
    nc = 2 * b
    cs = s // 2

    if _PROBE == "empty2":
        def body2(q_ref, out_ref):
            my_x = lax.axis_index("x")
            my_y = lax.axis_index("y")
            barrier_sem = pltpu.get_barrier_semaphore()
            for nb in ((my_x, 1 - my_y), (1 - my_x, my_y)):
                pl.semaphore_signal(
                    barrier_sem, inc=1, device_id=nb,
                    device_id_type=pl.DeviceIdType.MESH,
                )
            pl.semaphore_wait(barrier_sem, 2)
            out_ref[...] = jnp.zeros((b, s, hd), out_ref.dtype)

        out = pl.pallas_call(
            body2,
            out_shape=jax.ShapeDtypeStruct((b, s, hd), jnp.bfloat16),
            in_specs=[pl.BlockSpec(memory_space=pltpu.VMEM)],
            out_specs=pl.BlockSpec(memory_space=pltpu.VMEM),
            compiler_params=pltpu.CompilerParams(collective_id=0),
        )(Q.reshape(b, s, hd).astype(jnp.bfloat16))
        return out.reshape(b, s, h, d)

    def body(
        q_ref, k_ref, v_ref, out_ref,
        k_th, v_th, acc,
        p1s, p1r, p2s, p2r,
    ):
        my_x = lax.axis_index("x")
        my_y = lax.axis_index("y")
        ynbr = (my_x, 1 - my_y)
        xnbr = (1 - my_x, my_y)
        role_k = my_x == 0

        barrier_sem = pltpu.get_barrier_semaphore()
        for nb in (ynbr, xnbr):
            pl.semaphore_signal(
                barrier_sem, inc=1, device_id=nb,
                device_id_type=pl.DeviceIdType.MESH,
            )
        pl.semaphore_wait(barrier_sem, 2)

        def p1_desc(src, dst, c):
            bi, off = c // 2, (c % 2) * cs
            return pltpu.make_async_remote_copy(
                src_ref=src.at[bi, pl.ds(off, cs)],
                dst_ref=dst.at[bi, pl.ds(off, cs)],
                send_sem=p1s.at[c], recv_sem=p1r.at[c],
                device_id=ynbr, device_id_type=pl.DeviceIdType.MESH,
            )

        def p2_desc(T, c):
            bi, off = c // 2, (c % 2) * cs
            return pltpu.make_async_remote_copy(
                src_ref=T.at[bi, pl.ds(off, cs)],
                dst_ref=T.at[bi, pl.ds(off, cs)],
                send_sem=p2s.at[c], recv_sem=p2r.at[c],
                device_id=xnbr, device_id_type=pl.DeviceIdType.MESH,
            )

        if _PROBE == "compute":
            k_th[...] = k_ref[...]
            v_th[...] = v_ref[...]
        else:
            @pl.when(role_k)
            def _():
                for c in range(nc):
                    p1_desc(k_ref, k_th, c).start()

            @pl.when(jnp.logical_not(role_k))
            def _():
                for c in range(nc):
                    p1_desc(v_ref, v_th, c).start()

        ones = jnp.ones((s, d), jnp.bfloat16)

        def part_acc(kp_ref, vp_ref, bi, hi):
            sl = slice(hi * d, (hi + 1) * d)
            q = q_ref[bi, :, sl]
            kp = kp_ref[bi, :, sl]
            sc = lax.dot_general(
                q, kp, (((1,), (1,)), ((), ())),
                preferred_element_type=jnp.float32,
            )
            p = jnp.exp(sc.astype(jnp.bfloat16))
            va = jnp.concatenate(
                [vp_ref[bi, :, sl], ones], axis=1
            )
            return lax.dot_general(
                p, va, (((1,), (0,)), ((), ())),
                preferred_element_type=jnp.float32,
            )

        if _PROBE != "comm":
            for bi in range(b):
                for hi in range(h):
                    acc[bi, :, hi * da:(hi + 1) * da] = part_acc(
                        k_ref, v_ref, bi, hi
                    ).astype(acc.dtype)

        if _PROBE != "compute":
            @pl.when(role_k)
            def _():
                for c in range(nc):
                    p1_desc(k_ref, k_th, c).wait_recv()
                    p2_desc(k_th, c).start()

            @pl.when(jnp.logical_not(role_k))
            def _():
                for c in range(nc):
                    p1_desc(v_ref, v_th, c).wait_recv()
                    p2_desc(v_th, c).start()

        for bi in range(b):
            if _PROBE not in ("comm", "compute"):
                @pl.when(role_k)
                def _(bi=bi):
                    for c in (2 * bi, 2 * bi + 1):
                        p2_desc(v_th, c).wait_recv()

                @pl.when(jnp.logical_not(role_k))
                def _(bi=bi):
                    for c in (2 * bi, 2 * bi + 1):
                        p2_desc(k_th, c).wait_recv()

            if _PROBE != "comm":
                for hi in range(h):
                    a = acc[bi, :, hi * da:(hi + 1) * da] + part_acc(
                        k_th, v_th, bi, hi
                    )
                    r = 1.0 / a[:, d:d + 1]
                    out_ref[bi, :, hi * d:(hi + 1) * d] = (
                        a[:, :d] * r
                    ).astype(out_ref.dtype)

        if _PROBE == "comm":
            out_ref[...] = q_ref[...]
            for c in range(nc):
                @pl.when(role_k)
                def _(c=c):
                    p2_desc(v_th, c).wait_recv()

                @pl.when(jnp.logical_not(role_k))
                def _(c=c):
                    p2_desc(k_th, c).wait_recv()

        if _PROBE != "compute":
            @pl.when(role_k)
            def _():
                for c in range(nc):
                    p1_desc(k_ref, k_th, c).wait_send()
                    p2_desc(k_th, c).wait_send()

            @pl.when(jnp.logical_not(role_k))
            def _():
                for c in range(nc):
                    p1_desc(v_ref, v_th, c).wait_send()
                    p2_desc(v_th, c).wait_send()

    Qb = Q.reshape(b, s, hd).astype(jnp.bfloat16)
    Kb = (K.reshape(b, s, hd) * scale).astype(jnp.bfloat16)
    Vb = V.reshape(b, s, hd).astype(jnp.bfloat16)
    out = pl.pallas_call(
        body,
        out_shape=jax.ShapeDtypeStruct((b, s, hd), jnp.bfloat16),
        in_specs=[
            pl.BlockSpec(memory_space=pltpu.VMEM),
            pl.BlockSpec(memory_space=pltpu.VMEM),
            pl.BlockSpec(memory_space=pltpu.VMEM),
        ],
        out_specs=pl.BlockSpec(memory_space=pltpu.VMEM),
        scratch_shapes=[
            pltpu.VMEM((b, s, hd), jnp.bfloat16),
            pltpu.VMEM((b, s, hd), jnp.bfloat16),
            pltpu.VMEM((b, s, h * da), jnp.bfloat16),
            pltpu.SemaphoreType.DMA((nc,)),
            pltpu.SemaphoreType.DMA((nc,)),
            pltpu.SemaphoreType.DMA((nc,)),
            pltpu.SemaphoreType.DMA((nc,)),
        ],
        compiler_params=pltpu.CompilerParams(collective_id=0),
    )(Qb, Kb, Vb)
    return out.reshape(b, s, h, d)


# device time: 17642 ns/iter; 1.0468x vs baseline; 1.0468x over previous
import os

import jax
import jax.numpy as jnp
from jax import lax
from jax.experimental import pallas as pl
from jax.experimental.pallas import tpu as pltpu

_PROBE = os.environ.get("KERNEL_PROBE", "")


def kernel(Q, K, V):
    b, s, h, d = Q.shape
    hd = h * d
    da = 2 * d
    scale = d ** -0.5
    ncb = 4
    nc = ncb * b
    cs = s // ncb

    if _PROBE == "empty2":
        def body2(q_ref, out_ref):
            my_x = lax.axis_index("x")
            my_y = lax.axis_index("y")
            barrier_sem = pltpu.get_barrier_semaphore()
            for nb in ((my_x, 1 - my_y), (1 - my_x, my_y)):
                pl.semaphore_signal(
                    barrier_sem, inc=1, device_id=nb,
                    device_id_type=pl.DeviceIdType.MESH,
                )
            pl.semaphore_wait(barrier_sem, 2)
            out_ref[...] = jnp.zeros((b, s, hd), out_ref.dtype)

        out = pl.pallas_call(
            body2,
            out_shape=jax.ShapeDtypeStruct((b, s, hd), jnp.bfloat16),
            in_specs=[pl.BlockSpec(memory_space=pltpu.VMEM)],
            out_specs=pl.BlockSpec(memory_space=pltpu.VMEM),
            compiler_params=pltpu.CompilerParams(collective_id=0),
        )(Q.reshape(b, s, hd).astype(jnp.bfloat16))
        return out.reshape(b, s, h, d)

    def body(
        q_ref, k_ref, v_ref, out_ref,
        k_th, v_th, acc,
        p1s, p1r, p2s, p2r,
    ):
        my_x = lax.axis_index("x")
        my_y = lax.axis_index("y")
        ynbr = (my_x, 1 - my_y)
        xnbr = (1 - my_x, my_y)
        role_k = my_x == 0

        barrier_sem = pltpu.get_barrier_semaphore()
        for nb in (ynbr, xnbr):
            pl.semaphore_signal(
                barrier_sem, inc=1, device_id=nb,
                device_id_type=pl.DeviceIdType.MESH,
            )
        pl.semaphore_wait(barrier_sem, 2)

        def p1_desc(src, dst, c):
            bi, off = c // ncb, (c % ncb) * cs
            return pltpu.make_async_remote_copy(
                src_ref=src.at[bi, pl.ds(off, cs)],
                dst_ref=dst.at[bi, pl.ds(off, cs)],
                send_sem=p1s.at[c], recv_sem=p1r.at[c],
                device_id=ynbr, device_id_type=pl.DeviceIdType.MESH,
            )

        def p2_desc(T, c):
            bi, off = c // ncb, (c % ncb) * cs
            return pltpu.make_async_remote_copy(
                src_ref=T.at[bi, pl.ds(off, cs)],
                dst_ref=T.at[bi, pl.ds(off, cs)],
                send_sem=p2s.at[c], recv_sem=p2r.at[c],
                device_id=xnbr, device_id_type=pl.DeviceIdType.MESH,
            )

        if _PROBE == "compute":
            k_th[...] = k_ref[...]
            v_th[...] = v_ref[...]
        else:
            @pl.when(role_k)
            def _():
                for c in range(nc):
                    p1_desc(k_ref, k_th, c).start()

            @pl.when(jnp.logical_not(role_k))
            def _():
                for c in range(nc):
                    p1_desc(v_ref, v_th, c).start()

        ones = jnp.ones((s, d), jnp.bfloat16)

        def part_acc(kp_ref, vp_ref, bi, hi):
            sl = slice(hi * d, (hi + 1) * d)
            q = q_ref[bi, :, sl]
            kp = kp_ref[bi, :, sl]
            sc = lax.dot_general(
                q, kp, (((1,), (1,)), ((), ())),
                preferred_element_type=jnp.float32,
            )
            p = jnp.exp(sc.astype(jnp.bfloat16))
            va = jnp.concatenate(
                [vp_ref[bi, :, sl], ones], axis=1
            )
            return lax.dot_general(
                p, va, (((1,), (0,)), ((), ())),
                preferred_element_type=jnp.float32,
            )

        if _PROBE != "comm":
            for bi in range(b):
                for hi in range(h):
                    acc[bi, :, hi * da:(hi + 1) * da] = part_acc(
                        k_ref, v_ref, bi, hi
                    ).astype(acc.dtype)

        if _PROBE != "compute":
            @pl.when(role_k)
            def _():
                for c in range(nc):
                    p1_desc(k_ref, k_th, c).wait_recv()
                    p2_desc(k_th, c).start()

            @pl.when(jnp.logical_not(role_k))
            def _():
                for c in range(nc):
                    p1_desc(v_ref, v_th, c).wait_recv()
                    p2_desc(v_th, c).start()

        for bi in range(b):
            if _PROBE not in ("comm", "compute"):
                @pl.when(role_k)
                def _(bi=bi):
                    for c in range(ncb * bi, ncb * (bi + 1)):
                        p2_desc(v_th, c).wait_recv()

                @pl.when(jnp.logical_not(role_k))
                def _(bi=bi):
                    for c in range(ncb * bi, ncb * (bi + 1)):
                        p2_desc(k_th, c).wait_recv()

            if _PROBE != "comm":
                for hi in range(h):
                    a = acc[bi, :, hi * da:(hi + 1) * da] + part_acc(
                        k_th, v_th, bi, hi
                    )
                    r = 1.0 / a[:, d:d + 1]
                    out_ref[bi, :, hi * d:(hi + 1) * d] = (
                        a[:, :d] * r
                    ).astype(out_ref.dtype)

        if _PROBE == "comm":
            out_ref[...] = q_ref[...]
            for c in range(nc):
                @pl.when(role_k)
                def _(c=c):
                    p2_desc(v_th, c).wait_recv()

                @pl.when(jnp.logical_not(role_k))
                def _(c=c):
                    p2_desc(k_th, c).wait_recv()

        if _PROBE != "compute":
            @pl.when(role_k)
            def _():
                for c in range(nc):
                    p1_desc(k_ref, k_th, c).wait_send()
                    p2_desc(k_th, c).wait_send()

            @pl.when(jnp.logical_not(role_k))
            def _():
                for c in range(nc):
                    p1_desc(v_ref, v_th, c).wait_send()
                    p2_desc(v_th, c).wait_send()

    Qb = Q.reshape(b, s, hd).astype(jnp.bfloat16)
    Kb = (K.reshape(b, s, hd) * scale).astype(jnp.bfloat16)
    Vb = V.reshape(b, s, hd).astype(jnp.bfloat16)
    out = pl.pallas_call(
        body,
        out_shape=jax.ShapeDtypeStruct((b, s, hd), jnp.bfloat16),
        in_specs=[
            pl.BlockSpec(memory_space=pltpu.VMEM),
            pl.BlockSpec(memory_space=pltpu.VMEM),
            pl.BlockSpec(memory_space=pltpu.VMEM),
        ],
        out_specs=pl.BlockSpec(memory_space=pltpu.VMEM),
        scratch_shapes=[
            pltpu.VMEM((b, s, hd), jnp.bfloat16),
            pltpu.VMEM((b, s, hd), jnp.bfloat16),
            pltpu.VMEM((b, s, h * da), jnp.bfloat16),
            pltpu.SemaphoreType.DMA((nc,)),
            pltpu.SemaphoreType.DMA((nc,)),
            pltpu.SemaphoreType.DMA((nc,)),
            pltpu.SemaphoreType.DMA((nc,)),
        ],
        compiler_params=pltpu.CompilerParams(collective_id=0),
    )(Qb, Kb, Vb)
    return out.reshape(b, s, h, d)
